# baseline (speedup 1.0000x reference)
import jax
import jax.numpy as jnp
from jax.experimental import pallas as pl
from jax.experimental.pallas import tpu as pltpu


def _router_kernel(x_ref, w_ref, b_ref, logits_ref, probs_ref):
    x = x_ref[...]
    w = w_ref[...]
    logits = jax.lax.dot_general(
        w, x,
        dimension_numbers=(((1,), (1,)), ((), ())),
        preferred_element_type=jnp.float32,
    ) + b_ref[...]
    logits_ref[...] = logits
    m = jnp.max(logits, axis=0, keepdims=True)
    e = jnp.exp(logits - m)
    probs_ref[...] = e / jnp.sum(e, axis=0, keepdims=True)


def kernel(input, W, b):
    n, d = input.shape
    num_experts = W.shape[0]
    tile = 8192
    grid = (n // tile,)
    b2 = b.reshape(num_experts, 1)
    logits_t, probs_t = pl.pallas_call(
        _router_kernel,
        grid=grid,
        in_specs=[
            pl.BlockSpec((tile, d), lambda i: (i, 0)),
            pl.BlockSpec((num_experts, d), lambda i: (0, 0)),
            pl.BlockSpec((num_experts, 1), lambda i: (0, 0)),
        ],
        out_specs=[
            pl.BlockSpec((num_experts, tile), lambda i: (0, i)),
            pl.BlockSpec((num_experts, tile), lambda i: (0, i)),
        ],
        out_shape=[
            jax.ShapeDtypeStruct((num_experts, n), jnp.float32),
            jax.ShapeDtypeStruct((num_experts, n), jnp.float32),
        ],
        compiler_params=pltpu.CompilerParams(
            dimension_semantics=("parallel",),
        ),
    )(input, W, b2)
    return (logits_t.T, probs_t.T)


# speedup vs baseline: 1.0534x; 1.0534x over previous
import jax
import jax.numpy as jnp
from jax.experimental import pallas as pl
from jax.experimental.pallas import tpu as pltpu


def _router_kernel(x_ref, w_ref, b_ref, logits_ref, probs_ref):
    x = x_ref[...]
    w = w_ref[...]
    logits = jax.lax.dot_general(
        w, x,
        dimension_numbers=(((1,), (1,)), ((), ())),
        preferred_element_type=jnp.float32,
    ) + b_ref[...]
    logits_ref[...] = logits
    m = jnp.max(logits, axis=0, keepdims=True)
    e = jnp.exp(logits - m)
    probs_ref[...] = e / jnp.sum(e, axis=0, keepdims=True)


def kernel(input, W, b):
    n, d = input.shape
    num_experts = W.shape[0]
    tile = 4096
    grid = (n // tile,)
    b2 = b.reshape(num_experts, 1)
    logits_t, probs_t = pl.pallas_call(
        _router_kernel,
        grid=grid,
        in_specs=[
            pl.BlockSpec((tile, d), lambda i: (i, 0)),
            pl.BlockSpec((num_experts, d), lambda i: (0, 0)),
            pl.BlockSpec((num_experts, 1), lambda i: (0, 0)),
        ],
        out_specs=[
            pl.BlockSpec((num_experts, tile), lambda i: (0, i)),
            pl.BlockSpec((num_experts, tile), lambda i: (0, i)),
        ],
        out_shape=[
            jax.ShapeDtypeStruct((num_experts, n), jnp.float32),
            jax.ShapeDtypeStruct((num_experts, n), jnp.float32),
        ],
        compiler_params=pltpu.CompilerParams(
            dimension_semantics=("parallel",),
        ),
    )(input, W, b2)
    return (logits_t.T, probs_t.T)


# speedup vs baseline: 1.0555x; 1.0020x over previous
import jax
import jax.numpy as jnp
from jax.experimental import pallas as pl
from jax.experimental.pallas import tpu as pltpu


def _router_kernel(xa_ref, xb_ref, w_ref, b_ref, logits_ref, probs_ref):
    w = w_ref[...]
    half = xa_ref.shape[0]
    la = jax.lax.dot_general(
        w, xa_ref[...],
        dimension_numbers=(((1,), (1,)), ((), ())),
        preferred_element_type=jnp.float32,
    )
    lb = jax.lax.dot_general(
        w, xb_ref[...],
        dimension_numbers=(((1,), (1,)), ((), ())),
        preferred_element_type=jnp.float32,
    )
    logits = jnp.concatenate([la, lb], axis=1) + b_ref[...]
    logits_ref[...] = logits
    m = jnp.max(logits, axis=0, keepdims=True)
    e = jnp.exp(logits - m)
    probs_ref[...] = e / jnp.sum(e, axis=0, keepdims=True)


def kernel(input, W, b):
    n, d = input.shape
    num_experts = W.shape[0]
    tile = 4096
    half = tile // 2
    grid = (n // tile,)
    b2 = b.reshape(num_experts, 1)
    logits_t, probs_t = pl.pallas_call(
        _router_kernel,
        grid=grid,
        in_specs=[
            pl.BlockSpec((half, d), lambda i: (2 * i, 0)),
            pl.BlockSpec((half, d), lambda i: (2 * i + 1, 0)),
            pl.BlockSpec((num_experts, d), lambda i: (0, 0)),
            pl.BlockSpec((num_experts, 1), lambda i: (0, 0)),
        ],
        out_specs=[
            pl.BlockSpec((num_experts, tile), lambda i: (0, i)),
            pl.BlockSpec((num_experts, tile), lambda i: (0, i)),
        ],
        out_shape=[
            jax.ShapeDtypeStruct((num_experts, n), jnp.float32),
            jax.ShapeDtypeStruct((num_experts, n), jnp.float32),
        ],
        compiler_params=pltpu.CompilerParams(
            dimension_semantics=("parallel",),
        ),
    )(input, input, W, b2)
    return (logits_t.T, probs_t.T)
